# baseline (speedup 1.0000x reference)
import jax
import jax.numpy as jnp
from jax import lax
from jax.experimental import pallas as pl
from jax.experimental.pallas import tpu as pltpu
from jax.experimental.pallas import tpu_sc as plsc

E = 64
TOPK = 2
D = 1024
F = 1024
T = 64

_HI = lax.Precision.HIGHEST


def _logits_kernel(hid_ref, rw_ref, log_ref):
    log_ref[...] = lax.dot_general(
        hid_ref[...], rw_ref[...], (((1,), (1,)), ((), ())),
        preferred_element_type=jnp.float32)


def _route_sc_body(log_hbm, gsched_hbm, sched_hbm, nact_hbm,
                   lg_v, gsched_v, cnt_v, pos_v, sched_v, nact_v):
    c = lax.axis_index("c")
    s = lax.axis_index("s")

    @pl.when((c == 0) & (s == 0))
    def _():
        pltpu.sync_copy(log_hbm, lg_v)
        zero16 = jnp.zeros((16,), jnp.float32)
        iota16 = lax.iota(jnp.int32, 16)
        ones16 = jnp.ones((16,), jnp.float32)
        for k in range(4):
            cnt_v[pl.ds(16 * k, 16)] = zero16

        def zrow(t, carry):
            for k in range(4):
                gsched_v[t, pl.ds(16 * k, 16)] = zero16
            return carry
        lax.fori_loop(0, T, zrow, 0)

        neg = jnp.full((16,), -jnp.inf, jnp.float32)
        zi = jnp.zeros((16,), jnp.int32)
        tokv = [iota16 + 16 * b for b in range(4)]

        def body(e, carry):
            ev = zi + e
            out = []
            for b in range(4):
                m1, i1, m2, i2 = carry[b]
                lv = plsc.load_gather(lg_v, [tokv[b], ev])
                gt1 = lv > m1
                gt2 = lv > m2
                m2n = jnp.where(gt1, m1, jnp.where(gt2, lv, m2))
                i2n = jnp.where(gt1, i1, jnp.where(gt2, ev, i2))
                m1n = jnp.where(gt1, lv, m1)
                i1n = jnp.where(gt1, ev, i1)
                out.append((m1n, i1n, m2n, i2n))
            return tuple(out)

        init = tuple((neg, zi, neg, zi) for _ in range(4))
        res = lax.fori_loop(0, E, body, init)

        gw = []
        for b in range(4):
            m1, i1, m2, i2 = res[b]
            e2 = jnp.exp(m2 - m1)
            w1 = 1.0 / (1.0 + e2)
            w2 = e2 / (1.0 + e2)
            gw.append((i1, i2, w1, w2))
            plsc.store_scatter(cnt_v, [i1], ones16)
            plsc.store_scatter(cnt_v, [i2], ones16)

        carry = jnp.float32(0.0)
        for k in range(4):
            cv = cnt_v[pl.ds(16 * k, 16)]
            av = (cv > 0.0).astype(jnp.float32)
            incl = plsc.cumsum(av)
            excl = incl - av + carry
            posk = excl.astype(jnp.int32)
            pos_v[pl.ds(16 * k, 16)] = posk
            plsc.store_scatter(sched_v, [posk], iota16 + 16 * k,
                               mask=cv > 0.0)
            carry = carry + jnp.sum(av)
        nact_v[pl.ds(0, 16)] = zi + carry.astype(jnp.int32)

        for b in range(4):
            i1, i2, w1, w2 = gw[b]
            j1 = plsc.load_gather(pos_v, [i1])
            j2 = plsc.load_gather(pos_v, [i2])
            tok = tokv[b]
            plsc.store_scatter(gsched_v, [tok, j1], w1)
            plsc.store_scatter(gsched_v, [tok, j2], w2)

        pltpu.sync_copy(gsched_v, gsched_hbm)
        pltpu.sync_copy(sched_v, sched_hbm)
        pltpu.sync_copy(nact_v, nact_hbm)


def _route_sc(logits):
    return pl.kernel(
        _route_sc_body,
        out_type=[
            jax.ShapeDtypeStruct((T, E), jnp.float32),
            jax.ShapeDtypeStruct((E,), jnp.int32),
            jax.ShapeDtypeStruct((16,), jnp.int32),
        ],
        mesh=plsc.VectorSubcoreMesh(core_axis_name="c",
                                    subcore_axis_name="s"),
        compiler_params=pltpu.CompilerParams(needs_layout_passes=False),
        scratch_types=[
            pltpu.VMEM((T, E), jnp.float32),
            pltpu.VMEM((T, E), jnp.float32),
            pltpu.VMEM((E,), jnp.float32),
            pltpu.VMEM((E,), jnp.int32),
            pltpu.VMEM((E,), jnp.int32),
            pltpu.VMEM((16,), jnp.int32),
        ],
    )(logits)


def _moe_kernel(sched_sref, nact_sref, hid_ref, w13_ref, w2_ref, gsched_ref,
                out_ref):
    i = pl.program_id(0)

    @pl.when(i == 0)
    def _init():
        out_ref[...] = jnp.zeros_like(out_ref)

    @pl.when(i < nact_sref[0])
    def _compute():
        hid = hid_ref[...]
        h = lax.dot_general(hid, w13_ref[0], (((1,), (1,)), ((), ())),
                            preferred_element_type=jnp.float32)
        gatep = h[:, :F]
        up = h[:, F:]
        act = gatep * (1.0 / (1.0 + jnp.exp(-gatep))) * up
        y = lax.dot_general(act, w2_ref[0], (((1,), (1,)), ((), ())),
                            preferred_element_type=jnp.float32)
        onehot = (lax.broadcasted_iota(jnp.int32, (E, 1), 0) == i
                  ).astype(jnp.float32)
        g = lax.dot_general(gsched_ref[...], onehot,
                            (((1,), (0,)), ((), ())),
                            preferred_element_type=jnp.float32,
                            precision=_HI)
        out_ref[...] += y * g


def kernel(hidden_states, router_weight, w13, w2):
    logits = pl.pallas_call(
        _logits_kernel,
        out_shape=jax.ShapeDtypeStruct((T, E), jnp.float32),
    )(hidden_states, router_weight)

    gsched, sched, nact16 = _route_sc(logits)

    def _wsel(i, s, n):
        return (s[jnp.minimum(i, n[0] - 1)], 0, 0)

    grid_spec = pltpu.PrefetchScalarGridSpec(
        num_scalar_prefetch=2,
        grid=(E,),
        in_specs=[
            pl.BlockSpec((T, D), lambda i, s, n: (0, 0)),
            pl.BlockSpec((1, 2 * F, D), _wsel),
            pl.BlockSpec((1, D, F), _wsel),
            pl.BlockSpec((T, E), lambda i, s, n: (0, 0)),
        ],
        out_specs=pl.BlockSpec((T, D), lambda i, s, n: (0, 0)),
    )
    return pl.pallas_call(
        _moe_kernel,
        grid_spec=grid_spec,
        out_shape=jax.ShapeDtypeStruct((T, D), jnp.float32),
        compiler_params=pltpu.CompilerParams(
            dimension_semantics=("arbitrary",)),
    )(sched, nact16, hidden_states, w13, w2, gsched)


# speedup vs baseline: 1.0026x; 1.0026x over previous
import jax
import jax.numpy as jnp
from jax import lax
from jax.experimental import pallas as pl
from jax.experimental.pallas import tpu as pltpu
from jax.experimental.pallas import tpu_sc as plsc

E = 64
TOPK = 2
D = 1024
F = 1024
T = 64

_HI = lax.Precision.HIGHEST


def _logits_kernel(hid_ref, rw_ref, log_ref):
    log_ref[...] = lax.dot_general(
        hid_ref[...], rw_ref[...], (((1,), (1,)), ((), ())),
        preferred_element_type=jnp.float32)


def _route_sc_body(log_hbm, gsched_hbm, sched_hbm, nact_hbm,
                   lg_v, gsched_v, cnt_v, pos_v, sched_v, nact_v,
                   sem_in, sem_s, sem_n):
    c = lax.axis_index("c")
    s = lax.axis_index("s")

    @pl.when((c == 0) & (s == 0))
    def _():
        cp_in = pltpu.async_copy(log_hbm, lg_v, sem_in)
        zero16 = jnp.zeros((16,), jnp.float32)
        iota16 = lax.iota(jnp.int32, 16)
        ones16 = jnp.ones((16,), jnp.float32)
        for k in range(4):
            cnt_v[pl.ds(16 * k, 16)] = zero16

        def zrow(t, carry):
            for k in range(4):
                gsched_v[t, pl.ds(16 * k, 16)] = zero16
            return carry
        lax.fori_loop(0, T, zrow, 0)
        cp_in.wait()

        neg = jnp.full((16,), -jnp.inf, jnp.float32)
        zi = jnp.zeros((16,), jnp.int32)
        tokv = [iota16 + 16 * b for b in range(4)]

        def body(e, carry):
            ev = zi + e
            out = []
            for b in range(4):
                m1, i1, m2, i2 = carry[b]
                lv = plsc.load_gather(lg_v, [tokv[b], ev])
                gt1 = lv > m1
                gt2 = lv > m2
                m2n = jnp.where(gt1, m1, jnp.where(gt2, lv, m2))
                i2n = jnp.where(gt1, i1, jnp.where(gt2, ev, i2))
                m1n = jnp.where(gt1, lv, m1)
                i1n = jnp.where(gt1, ev, i1)
                out.append((m1n, i1n, m2n, i2n))
            return tuple(out)

        init = tuple((neg, zi, neg, zi) for _ in range(4))
        res = lax.fori_loop(0, E, body, init)

        gw = []
        for b in range(4):
            m1, i1, m2, i2 = res[b]
            e2 = jnp.exp(m2 - m1)
            w1 = 1.0 / (1.0 + e2)
            w2 = e2 / (1.0 + e2)
            gw.append((i1, i2, w1, w2))
            plsc.store_scatter(cnt_v, [i1], ones16)
            plsc.store_scatter(cnt_v, [i2], ones16)

        carry = jnp.float32(0.0)
        for k in range(4):
            cv = cnt_v[pl.ds(16 * k, 16)]
            av = (cv > 0.0).astype(jnp.float32)
            incl = plsc.cumsum(av)
            excl = incl - av + carry
            posk = excl.astype(jnp.int32)
            pos_v[pl.ds(16 * k, 16)] = posk
            plsc.store_scatter(sched_v, [posk], iota16 + 16 * k,
                               mask=cv > 0.0)
            carry = carry + jnp.sum(av)
        nact_v[pl.ds(0, 16)] = zi + carry.astype(jnp.int32)
        cp_s = pltpu.async_copy(sched_v, sched_hbm, sem_s)
        cp_n = pltpu.async_copy(nact_v, nact_hbm, sem_n)

        for b in range(4):
            i1, i2, w1, w2 = gw[b]
            j1 = plsc.load_gather(pos_v, [i1])
            j2 = plsc.load_gather(pos_v, [i2])
            tok = tokv[b]
            plsc.store_scatter(gsched_v, [tok, j1], w1)
            plsc.store_scatter(gsched_v, [tok, j2], w2)

        pltpu.sync_copy(gsched_v, gsched_hbm)
        cp_s.wait()
        cp_n.wait()


def _route_sc(logits):
    return pl.kernel(
        _route_sc_body,
        out_type=[
            jax.ShapeDtypeStruct((T, E), jnp.float32),
            jax.ShapeDtypeStruct((E,), jnp.int32),
            jax.ShapeDtypeStruct((16,), jnp.int32),
        ],
        mesh=plsc.VectorSubcoreMesh(core_axis_name="c",
                                    subcore_axis_name="s"),
        compiler_params=pltpu.CompilerParams(needs_layout_passes=False),
        scratch_types=[
            pltpu.VMEM((T, E), jnp.float32),
            pltpu.VMEM((T, E), jnp.float32),
            pltpu.VMEM((E,), jnp.float32),
            pltpu.VMEM((E,), jnp.int32),
            pltpu.VMEM((E,), jnp.int32),
            pltpu.VMEM((16,), jnp.int32),
            pltpu.SemaphoreType.DMA,
            pltpu.SemaphoreType.DMA,
            pltpu.SemaphoreType.DMA,
        ],
    )(logits)


def _moe_kernel(sched_sref, nact_sref, hid_ref, w13_ref, w2_ref, gsched_ref,
                out_ref):
    i = pl.program_id(0)

    @pl.when(i == 0)
    def _init():
        out_ref[...] = jnp.zeros_like(out_ref)

    @pl.when(i < nact_sref[0])
    def _compute():
        hid = hid_ref[...]
        h = lax.dot_general(hid, w13_ref[0], (((1,), (1,)), ((), ())),
                            preferred_element_type=jnp.float32)
        gatep = h[:, :F]
        up = h[:, F:]
        act = gatep * (1.0 / (1.0 + jnp.exp(-gatep))) * up
        y = lax.dot_general(act, w2_ref[0], (((1,), (1,)), ((), ())),
                            preferred_element_type=jnp.float32)
        onehot = (lax.broadcasted_iota(jnp.int32, (E, 1), 0) == i
                  ).astype(jnp.float32)
        g = lax.dot_general(gsched_ref[...], onehot,
                            (((1,), (0,)), ((), ())),
                            preferred_element_type=jnp.float32,
                            precision=_HI)
        out_ref[...] += y * g


def kernel(hidden_states, router_weight, w13, w2):
    logits = pl.pallas_call(
        _logits_kernel,
        out_shape=jax.ShapeDtypeStruct((T, E), jnp.float32),
    )(hidden_states, router_weight)

    gsched, sched, nact16 = _route_sc(logits)

    def _wsel(i, s, n):
        return (s[jnp.minimum(i, n[0] - 1)], 0, 0)

    grid_spec = pltpu.PrefetchScalarGridSpec(
        num_scalar_prefetch=2,
        grid=(E,),
        in_specs=[
            pl.BlockSpec((T, D), lambda i, s, n: (0, 0)),
            pl.BlockSpec((1, 2 * F, D), _wsel),
            pl.BlockSpec((1, D, F), _wsel),
            pl.BlockSpec((T, E), lambda i, s, n: (0, 0)),
        ],
        out_specs=pl.BlockSpec((T, D), lambda i, s, n: (0, 0)),
    )
    return pl.pallas_call(
        _moe_kernel,
        grid_spec=grid_spec,
        out_shape=jax.ShapeDtypeStruct((T, D), jnp.float32),
        compiler_params=pltpu.CompilerParams(
            dimension_semantics=("arbitrary",)),
    )(sched, nact16, hidden_states, w13, w2, gsched)
